# baseline (speedup 1.0000x reference)
import functools

import jax
import jax.numpy as jnp
import numpy as np
from jax import lax
from jax.experimental import pallas as pl
from jax.experimental.pallas import tpu as pltpu
from jax.experimental.pallas import tpu_sc as plsc

EPS = 1e-5
NC = 2
NS = 16
L = 16


def _sc_mesh():
    return plsc.VectorSubcoreMesh(core_axis_name="c", subcore_axis_name="s")


_SC_PARAMS = pltpu.CompilerParams(use_tc_tiling_on_sc=False)


def _zero_rows(buf, nrows, width):
    per_row = width // L

    def body(i, _):
        r = i // per_row
        d = i % per_row
        buf[r, pl.ds(d * L, L)] = jnp.zeros((L,), jnp.float32)
        return 0

    lax.fori_loop(0, nrows * per_row, body, 0)


def _zero_shared_rows(shared, vbuf, s, n_nodes, bufrows, width):
    rows_per_sub = n_nodes // NS
    nz = min(bufrows, rows_per_sub)
    _zero_rows(vbuf, nz, width)
    done = 0
    while done < rows_per_sub:
        step = min(nz, rows_per_sub - done)
        pltpu.sync_copy(vbuf.at[pl.ds(0, step)],
                        shared.at[pl.ds(s * rows_per_sub + done, step)])
        done += step


def _dump_rows(shared, vbuf, out_ref, c, s, n_nodes, bufrows):
    base8 = (n_nodes // NS) // 8 * 8
    tail = n_nodes - NS * base8
    cap = bufrows // 8 * 8
    start = s * base8
    done = 0
    while done < base8:
        step = min(cap, base8 - done)
        pltpu.sync_copy(shared.at[pl.ds(start + done, step)],
                        vbuf.at[pl.ds(0, step)])
        pltpu.sync_copy(vbuf.at[pl.ds(0, step)],
                        out_ref.at[c, pl.ds(start + done, step)])
        done += step
    if tail:
        @pl.when(s == NS - 1)
        def _():
            pltpu.sync_copy(shared.at[pl.ds(NS * base8, tail)],
                            vbuf.at[pl.ds(0, tail)])
            pltpu.sync_copy(vbuf.at[pl.ds(0, tail)],
                            out_ref.at[c, pl.ds(NS * base8, tail)])


def _deg_kernel(n_nodes, n_edges, ck, col_ref, ew_ref, out_ref,
                col_v, ew_v, wbuf, deg_sh):
    c = lax.axis_index("c")
    s = lax.axis_index("s")
    rows_per_sub = n_nodes // NS
    epw = n_edges // (NC * NS)
    nchunks = epw // ck
    ebase = c * (n_edges // NC) + s * epw

    _zero_shared_rows(deg_sh, wbuf, s, n_nodes, ck, L)
    plsc.subcore_barrier()

    def chunk(k, _):
        base = ebase + k * ck
        pltpu.sync_copy(col_ref.at[pl.ds(base, ck)], col_v)
        pltpu.sync_copy(ew_ref.at[pl.ds(base, ck)], ew_v)

        @plsc.parallel_loop(0, ck // L, unroll=2)
        def _fill(gi):
            wvec = ew_v[pl.ds(gi * L, L)]
            for lane in range(L):
                e = gi * L + lane
                wbuf[e, pl.ds(0, L)] = jnp.full((L,), wvec[lane], jnp.float32)
        pltpu.sync_copy(wbuf.at[pl.ds(0, ck)], deg_sh.at[col_v], add=True)
        return 0

    lax.fori_loop(0, nchunks, chunk, 0)
    plsc.subcore_barrier()

    _dump_rows(deg_sh, wbuf, out_ref, c, s, n_nodes, ck)


def _deg_call(col, edge_attr, n_nodes):
    n_edges = col.shape[0]
    ck = 2000
    body = functools.partial(_deg_kernel, n_nodes, n_edges, ck)
    return pl.kernel(
        body,
        out_type=jax.ShapeDtypeStruct((NC, n_nodes, L), jnp.float32),
        mesh=_sc_mesh(),
        scratch_types=[
            pltpu.VMEM((ck,), jnp.int32),
            pltpu.VMEM((ck,), jnp.float32),
            pltpu.VMEM((ck, L), jnp.float32),
            pltpu.VMEM_SHARED((n_nodes, L), jnp.float32),
        ],
        compiler_params=_SC_PARAMS,
    )(col, edge_attr)


_NR = 2
_NI = 3


def _spmm_kernel(n_nodes, n_edges, d, ck, row_ref, col_ref, ew_ref, y_ref,
                 out_ref, *scratch):
    row_v = scratch[0:_NI]
    col_v = scratch[_NI:2 * _NI]
    ew_v = scratch[2 * _NI:3 * _NI]
    rows_v = scratch[3 * _NI:3 * _NI + _NR]
    acc_sh = scratch[3 * _NI + _NR]
    si = scratch[3 * _NI + _NR + 1:4 * _NI + _NR + 1]
    sg = scratch[4 * _NI + _NR + 1:4 * _NI + 2 * _NR + 1]
    sc = scratch[4 * _NI + 2 * _NR + 1:4 * _NI + 3 * _NR + 1]

    c = lax.axis_index("c")
    s = lax.axis_index("s")
    epw = n_edges // (NC * NS)
    nchunks = epw // ck
    ebase = c * (n_edges // NC) + s * epw
    nd = d // L

    def idx_copies(k, b):
        base = ebase + k * ck
        return [
            pltpu.make_async_copy(row_ref.at[pl.ds(base, ck)], row_v[b], si[b]),
            pltpu.make_async_copy(col_ref.at[pl.ds(base, ck)], col_v[b], si[b]),
            pltpu.make_async_copy(ew_ref.at[pl.ds(base, ck)], ew_v[b], si[b]),
        ]

    def gather_copy(bi, br):
        return pltpu.make_async_copy(y_ref.at[row_v[bi]], rows_v[br], sg[br])

    def scatter_start(bi, br):
        pltpu.async_copy(rows_v[br], acc_sh.at[col_v[bi]], sc[br], add=True)

    def scatter_wait(bi, br):
        pltpu.make_async_copy(rows_v[br], acc_sh.at[col_v[bi]], sc[br]).wait()

    _zero_shared_rows(acc_sh, rows_v[0], s, n_nodes, ck, d)
    plsc.subcore_barrier()

    for cp in idx_copies(0, 0):
        cp.start()
    for cp in idx_copies(1, 1):
        cp.start()
    for cp in idx_copies(0, 0):
        cp.wait()
    gather_copy(0, 0).start()

    def slot(k, b2, b3):
        gather_copy(b3, b2).wait()
        @pl.when(k >= 1)
        def _():
            scatter_wait((b3 + 2) % _NI, (b2 + 1) % _NR)
        @pl.when(k + 1 < nchunks)
        def _():
            for cp in idx_copies(0, (b3 + 1) % _NI):
                cp.wait()
            gather_copy((b3 + 1) % _NI, (b2 + 1) % _NR).start()

        @plsc.parallel_loop(0, ck // L, unroll=4)
        def _scale(gi):
            wvec = ew_v[b3][pl.ds(gi * L, L)]
            for lane in range(L):
                e = gi * L + lane
                w = wvec[lane]
                for dd in range(nd):
                    sl = rows_v[b2][e, pl.ds(dd * L, L)]
                    rows_v[b2][e, pl.ds(dd * L, L)] = sl * w
        scatter_start(b3, b2)
        @pl.when(k + 2 < nchunks)
        def _():
            for cp in idx_copies(k + 2, (b3 + 2) % _NI):
                cp.start()

    def outer(g, _):
        for u in range(6):
            k = g * 6 + u

            @pl.when(k < nchunks)
            def _():
                slot(k, u % _NR, u % _NI)
            del _
        return 0

    lax.fori_loop(0, (nchunks + 5) // 6, outer, 0)
    scatter_wait((nchunks - 1) % _NI, (nchunks - 1) % _NR)
    plsc.subcore_barrier()

    _dump_rows(acc_sh, rows_v[0], out_ref, c, s, n_nodes, ck)


def _spmm_call(row, col, edge_attr, y):
    n_nodes, d = y.shape
    n_edges = row.shape[0]
    ck = 400
    body = functools.partial(_spmm_kernel, n_nodes, n_edges, d, ck)
    return pl.kernel(
        body,
        out_type=jax.ShapeDtypeStruct((NC, n_nodes, d), jnp.float32),
        mesh=_sc_mesh(),
        scratch_types=(
            [pltpu.VMEM((ck,), jnp.int32)] * _NI
            + [pltpu.VMEM((ck,), jnp.int32)] * _NI
            + [pltpu.VMEM((ck,), jnp.float32)] * _NI
            + [pltpu.VMEM((ck, d), jnp.float32)] * _NR
            + [pltpu.VMEM_SHARED((n_nodes, d), jnp.float32)]
            + [pltpu.SemaphoreType.DMA] * _NI
            + [pltpu.SemaphoreType.DMA] * (2 * _NR)
        ),
        compiler_params=_SC_PARAMS,
    )(row, col, edge_attr, y)


_BLK = 1000


def _tc1_body(x_ref, w1_ref, degp_ref, y1_ref, dinv_ref):
    deg = degp_ref[0] + degp_ref[1] + 1.0
    dinv = lax.rsqrt(deg)
    xw = jnp.dot(x_ref[...], w1_ref[...], preferred_element_type=jnp.float32)
    y1_ref[...] = xw * dinv[:, 0:1]
    dinv_ref[...] = dinv


def _tc1_call(x, w1, degp):
    n, din = x.shape
    dh = w1.shape[1]
    grid = n // _BLK
    return pl.pallas_call(
        _tc1_body,
        grid=(grid,),
        in_specs=[
            pl.BlockSpec((_BLK, din), lambda i: (i, 0)),
            pl.BlockSpec((din, dh), lambda i: (0, 0)),
            pl.BlockSpec((NC, _BLK, L), lambda i: (0, i, 0)),
        ],
        out_specs=[
            pl.BlockSpec((_BLK, dh), lambda i: (i, 0)),
            pl.BlockSpec((_BLK, L), lambda i: (i, 0)),
        ],
        out_shape=[
            jax.ShapeDtypeStruct((n, dh), jnp.float32),
            jax.ShapeDtypeStruct((n, L), jnp.float32),
        ],
    )(x, w1, degp)


def _tc2_body(acc_ref, y1_ref, dinv_ref, w2_ref, b1_ref, g1_ref, bb1_ref,
              y2_ref):
    s = acc_ref[0] + acc_ref[1] + y1_ref[...]
    dinv = dinv_ref[...][:, 0:1]
    kbn = g1_ref[...] * np.float32(1.0 / np.sqrt(1.0 + EPS))
    h = (s * dinv + b1_ref[...]) * kbn + bb1_ref[...]
    h = jnp.maximum(h, 0.0)
    y2_ref[...] = jnp.dot(h, w2_ref[...],
                          preferred_element_type=jnp.float32) * dinv


def _tc2_call(acc, y1, dinv, w2, b1, g1, bb1):
    n, dh = y1.shape
    grid = n // _BLK
    vec = lambda a: a.reshape(1, -1)
    return pl.pallas_call(
        _tc2_body,
        grid=(grid,),
        in_specs=[
            pl.BlockSpec((NC, _BLK, dh), lambda i: (0, i, 0)),
            pl.BlockSpec((_BLK, dh), lambda i: (i, 0)),
            pl.BlockSpec((_BLK, L), lambda i: (i, 0)),
            pl.BlockSpec((dh, dh), lambda i: (0, 0)),
            pl.BlockSpec((1, dh), lambda i: (0, 0)),
            pl.BlockSpec((1, dh), lambda i: (0, 0)),
            pl.BlockSpec((1, dh), lambda i: (0, 0)),
        ],
        out_specs=pl.BlockSpec((_BLK, dh), lambda i: (i, 0)),
        out_shape=jax.ShapeDtypeStruct((n, dh), jnp.float32),
    )(acc, y1, dinv, w2, vec(b1), vec(g1), vec(bb1))


def _tc3_body(n_graphs, acc_ref, y2_ref, dinv_ref, batch_ref, b2_ref, g2_ref,
              bb2_ref, wout_ref, bout_ref, out_ref, pooled_acc, cnt_acc):
    i = pl.program_id(0)
    s = acc_ref[0] + acc_ref[1] + y2_ref[...]
    dinv = dinv_ref[...][:, 0:1]
    kbn = g2_ref[...] * np.float32(1.0 / np.sqrt(1.0 + EPS))
    h = (s * dinv + b2_ref[...]) * kbn + bb2_ref[...]
    h = jnp.maximum(h, 0.0)
    gids = lax.broadcasted_iota(jnp.int32, (1, n_graphs), 1)
    onehot = (batch_ref[...] == gids).astype(jnp.float32)
    dh = h.shape[1]
    pooled_p = lax.dot_general(onehot, h, (((0,), (0,)), ((), ())),
                               preferred_element_type=jnp.float32)
    cnt_p = lax.dot_general(onehot, jnp.ones_like(h), (((0,), (0,)), ((), ())),
                            preferred_element_type=jnp.float32)

    @pl.when(i == 0)
    def _():
        pooled_acc[...] = jnp.zeros_like(pooled_acc)
        cnt_acc[...] = jnp.zeros_like(cnt_acc)

    pooled_acc[...] += pooled_p
    cnt_acc[...] += cnt_p

    @pl.when(i == pl.num_programs(0) - 1)
    def _():
        pooled = pooled_acc[...] / jnp.maximum(cnt_acc[...], 1.0)
        out_ref[...] = jnp.dot(pooled, wout_ref[...],
                               preferred_element_type=jnp.float32) + bout_ref[...]


def _tc3_call(acc, y2, dinv, batch, b2, g2, bb2, wout, bout, n_graphs):
    n, dh = y2.shape
    dout = wout.shape[1]
    grid = n // _BLK
    vec = lambda a: a.reshape(1, -1)
    body = functools.partial(_tc3_body, n_graphs)
    return pl.pallas_call(
        body,
        grid=(grid,),
        in_specs=[
            pl.BlockSpec((NC, _BLK, dh), lambda i: (0, i, 0)),
            pl.BlockSpec((_BLK, dh), lambda i: (i, 0)),
            pl.BlockSpec((_BLK, L), lambda i: (i, 0)),
            pl.BlockSpec((_BLK, 1), lambda i: (i, 0)),
            pl.BlockSpec((1, dh), lambda i: (0, 0)),
            pl.BlockSpec((1, dh), lambda i: (0, 0)),
            pl.BlockSpec((1, dh), lambda i: (0, 0)),
            pl.BlockSpec((dh, dout), lambda i: (0, 0)),
            pl.BlockSpec((1, dout), lambda i: (0, 0)),
        ],
        out_specs=pl.BlockSpec((n_graphs, dout), lambda i: (0, 0)),
        out_shape=jax.ShapeDtypeStruct((n_graphs, dout), jnp.float32),
        scratch_shapes=[
            pltpu.VMEM((n_graphs, dh), jnp.float32),
            pltpu.VMEM((n_graphs, dh), jnp.float32),
        ],
    )(acc, y2, dinv, batch.reshape(-1, 1), vec(b2), vec(g2), vec(bb2),
      wout, bout.reshape(1, -1))


def kernel(x, edge_index, edge_attr, batch, W1, b1, bn1_g, bn1_b,
           W2, b2, bn2_g, bn2_b, Wout, bout):
    n = x.shape[0]
    g = 16
    edge_index = edge_index.astype(jnp.int32)
    batch = batch.astype(jnp.int32)
    row = edge_index[0]
    col = edge_index[1]

    degp = _deg_call(col, edge_attr, n)
    y1, dinv = _tc1_call(x, W1, degp)
    acc1 = _spmm_call(row, col, edge_attr, y1)
    y2 = _tc2_call(acc1, y1, dinv, W2, b1, bn1_g, bn1_b)
    acc2 = _spmm_call(row, col, edge_attr, y2)
    return _tc3_call(acc2, y2, dinv, batch, b2, bn2_g, bn2_b,
                     Wout, bout, g)


# speedup vs baseline: 1.0096x; 1.0096x over previous
import functools

import jax
import jax.numpy as jnp
import numpy as np
from jax import lax
from jax.experimental import pallas as pl
from jax.experimental.pallas import tpu as pltpu
from jax.experimental.pallas import tpu_sc as plsc

EPS = 1e-5
NC = 2
NS = 16
L = 16


def _sc_mesh():
    return plsc.VectorSubcoreMesh(core_axis_name="c", subcore_axis_name="s")


_SC_PARAMS = pltpu.CompilerParams(use_tc_tiling_on_sc=False)


def _zero_rows(buf, nrows, width):
    per_row = width // L

    def body(i, _):
        r = i // per_row
        d = i % per_row
        buf[r, pl.ds(d * L, L)] = jnp.zeros((L,), jnp.float32)
        return 0

    lax.fori_loop(0, nrows * per_row, body, 0)


def _zero_shared_rows(shared, vbuf, s, n_nodes, bufrows, width):
    rows_per_sub = n_nodes // NS
    nz = min(bufrows, rows_per_sub)
    _zero_rows(vbuf, nz, width)
    done = 0
    while done < rows_per_sub:
        step = min(nz, rows_per_sub - done)
        pltpu.sync_copy(vbuf.at[pl.ds(0, step)],
                        shared.at[pl.ds(s * rows_per_sub + done, step)])
        done += step


def _dump_rows(shared, vbuf, out_ref, c, s, n_nodes, bufrows):
    base8 = (n_nodes // NS) // 8 * 8
    tail = n_nodes - NS * base8
    cap = bufrows // 8 * 8
    start = s * base8
    done = 0
    while done < base8:
        step = min(cap, base8 - done)
        pltpu.sync_copy(shared.at[pl.ds(start + done, step)],
                        vbuf.at[pl.ds(0, step)])
        pltpu.sync_copy(vbuf.at[pl.ds(0, step)],
                        out_ref.at[c, pl.ds(start + done, step)])
        done += step
    if tail:
        @pl.when(s == NS - 1)
        def _():
            pltpu.sync_copy(shared.at[pl.ds(NS * base8, tail)],
                            vbuf.at[pl.ds(0, tail)])
            pltpu.sync_copy(vbuf.at[pl.ds(0, tail)],
                            out_ref.at[c, pl.ds(NS * base8, tail)])


def _deg_kernel(n_nodes, n_edges, ck, col_ref, ew_ref, out_ref,
                col_v, ew_v, wbuf, deg_sh):
    c = lax.axis_index("c")
    s = lax.axis_index("s")
    rows_per_sub = n_nodes // NS
    epw = n_edges // (NC * NS)
    nchunks = epw // ck
    ebase = c * (n_edges // NC) + s * epw

    _zero_shared_rows(deg_sh, wbuf, s, n_nodes, ck, L)
    plsc.subcore_barrier()

    def chunk(k, _):
        base = ebase + k * ck
        pltpu.sync_copy(col_ref.at[pl.ds(base, ck)], col_v)
        pltpu.sync_copy(ew_ref.at[pl.ds(base, ck)], ew_v)

        @plsc.parallel_loop(0, ck // L, unroll=2)
        def _fill(gi):
            wvec = ew_v[pl.ds(gi * L, L)]
            for lane in range(L):
                e = gi * L + lane
                wbuf[e, pl.ds(0, L)] = jnp.full((L,), wvec[lane], jnp.float32)
        pltpu.sync_copy(wbuf.at[pl.ds(0, ck)], deg_sh.at[col_v], add=True)
        return 0

    lax.fori_loop(0, nchunks, chunk, 0)
    plsc.subcore_barrier()

    _dump_rows(deg_sh, wbuf, out_ref, c, s, n_nodes, ck)


def _deg_call(col, edge_attr, n_nodes):
    n_edges = col.shape[0]
    ck = 2000
    body = functools.partial(_deg_kernel, n_nodes, n_edges, ck)
    return pl.kernel(
        body,
        out_type=jax.ShapeDtypeStruct((NC, n_nodes, L), jnp.float32),
        mesh=_sc_mesh(),
        scratch_types=[
            pltpu.VMEM((ck,), jnp.int32),
            pltpu.VMEM((ck,), jnp.float32),
            pltpu.VMEM((ck, L), jnp.float32),
            pltpu.VMEM_SHARED((n_nodes, L), jnp.float32),
        ],
        compiler_params=_SC_PARAMS,
    )(col, edge_attr)


_NR = 2
_NI = 3


def _spmm_kernel(n_nodes, n_edges, d, ck, row_ref, col_ref, ew_ref, y_ref,
                 out_ref, *scratch):
    row_v = scratch[0:_NI]
    col_v = scratch[_NI:2 * _NI]
    ew_v = scratch[2 * _NI:3 * _NI]
    rows_v = scratch[3 * _NI:3 * _NI + _NR]
    acc_sh = scratch[3 * _NI + _NR]
    si = scratch[3 * _NI + _NR + 1:4 * _NI + _NR + 1]
    sg = scratch[4 * _NI + _NR + 1:4 * _NI + 2 * _NR + 1]
    sc = scratch[4 * _NI + 2 * _NR + 1:4 * _NI + 3 * _NR + 1]

    c = lax.axis_index("c")
    s = lax.axis_index("s")
    epw = n_edges // (NC * NS)
    nchunks = epw // ck
    ebase = c * (n_edges // NC) + s * epw
    nd = d // L

    def idx_copies(k, b):
        base = ebase + k * ck
        return [
            pltpu.make_async_copy(row_ref.at[pl.ds(base, ck)], row_v[b], si[b]),
            pltpu.make_async_copy(col_ref.at[pl.ds(base, ck)], col_v[b], si[b]),
            pltpu.make_async_copy(ew_ref.at[pl.ds(base, ck)], ew_v[b], si[b]),
        ]

    def gather_copy(bi, br):
        return pltpu.make_async_copy(y_ref.at[row_v[bi]], rows_v[br], sg[br])

    def scatter_start(bi, br):
        pltpu.async_copy(rows_v[br], acc_sh.at[col_v[bi]], sc[br], add=True)

    def scatter_wait(bi, br):
        pltpu.make_async_copy(rows_v[br], acc_sh.at[col_v[bi]], sc[br]).wait()

    _zero_shared_rows(acc_sh, rows_v[0], s, n_nodes, ck, d)
    plsc.subcore_barrier()

    for cp in idx_copies(0, 0):
        cp.start()
    for cp in idx_copies(1, 1):
        cp.start()
    for cp in idx_copies(0, 0):
        cp.wait()
    gather_copy(0, 0).start()

    def slot(k, b2, b3):
        gather_copy(b3, b2).wait()
        @pl.when(k >= 1)
        def _():
            scatter_wait((b3 + 2) % _NI, (b2 + 1) % _NR)
        @pl.when(k + 1 < nchunks)
        def _():
            for cp in idx_copies(0, (b3 + 1) % _NI):
                cp.wait()
            gather_copy((b3 + 1) % _NI, (b2 + 1) % _NR).start()

        @plsc.parallel_loop(0, ck // L, unroll=2)
        def _scale(gi):
            wvec = ew_v[b3][pl.ds(gi * L, L)]
            for lane in range(L):
                e = gi * L + lane
                w = wvec[lane]
                for dd in range(nd):
                    sl = rows_v[b2][e, pl.ds(dd * L, L)]
                    rows_v[b2][e, pl.ds(dd * L, L)] = sl * w
        scatter_start(b3, b2)
        @pl.when(k + 2 < nchunks)
        def _():
            for cp in idx_copies(k + 2, (b3 + 2) % _NI):
                cp.start()

    def outer(g, _):
        for u in range(6):
            k = g * 6 + u

            @pl.when(k < nchunks)
            def _():
                slot(k, u % _NR, u % _NI)
            del _
        return 0

    lax.fori_loop(0, (nchunks + 5) // 6, outer, 0)
    scatter_wait((nchunks - 1) % _NI, (nchunks - 1) % _NR)
    plsc.subcore_barrier()

    _dump_rows(acc_sh, rows_v[0], out_ref, c, s, n_nodes, ck)


def _spmm_call(row, col, edge_attr, y):
    n_nodes, d = y.shape
    n_edges = row.shape[0]
    ck = 400
    body = functools.partial(_spmm_kernel, n_nodes, n_edges, d, ck)
    return pl.kernel(
        body,
        out_type=jax.ShapeDtypeStruct((NC, n_nodes, d), jnp.float32),
        mesh=_sc_mesh(),
        scratch_types=(
            [pltpu.VMEM((ck,), jnp.int32)] * _NI
            + [pltpu.VMEM((ck,), jnp.int32)] * _NI
            + [pltpu.VMEM((ck,), jnp.float32)] * _NI
            + [pltpu.VMEM((ck, d), jnp.float32)] * _NR
            + [pltpu.VMEM_SHARED((n_nodes, d), jnp.float32)]
            + [pltpu.SemaphoreType.DMA] * _NI
            + [pltpu.SemaphoreType.DMA] * (2 * _NR)
        ),
        compiler_params=_SC_PARAMS,
    )(row, col, edge_attr, y)


_BLK = 1000


def _tc1a_body(x_ref, w1_ref, xw_ref):
    xw_ref[...] = jnp.dot(x_ref[...], w1_ref[...],
                          preferred_element_type=jnp.float32)


def _tc1a_call(x, w1):
    n, din = x.shape
    dh = w1.shape[1]
    grid = n // _BLK
    return pl.pallas_call(
        _tc1a_body,
        grid=(grid,),
        in_specs=[
            pl.BlockSpec((_BLK, din), lambda i: (i, 0)),
            pl.BlockSpec((din, dh), lambda i: (0, 0)),
        ],
        out_specs=pl.BlockSpec((_BLK, dh), lambda i: (i, 0)),
        out_shape=jax.ShapeDtypeStruct((n, dh), jnp.float32),
    )(x, w1)


def _tc1b_body(xw_ref, degp_ref, y1_ref, dinv_ref):
    deg = degp_ref[0] + degp_ref[1] + 1.0
    dinv = lax.rsqrt(deg)
    y1_ref[...] = xw_ref[...] * dinv[:, 0:1]
    dinv_ref[...] = dinv


def _tc1b_call(xw, degp):
    n, dh = xw.shape
    grid = n // _BLK
    return pl.pallas_call(
        _tc1b_body,
        grid=(grid,),
        in_specs=[
            pl.BlockSpec((_BLK, dh), lambda i: (i, 0)),
            pl.BlockSpec((NC, _BLK, L), lambda i: (0, i, 0)),
        ],
        out_specs=[
            pl.BlockSpec((_BLK, dh), lambda i: (i, 0)),
            pl.BlockSpec((_BLK, L), lambda i: (i, 0)),
        ],
        out_shape=[
            jax.ShapeDtypeStruct((n, dh), jnp.float32),
            jax.ShapeDtypeStruct((n, L), jnp.float32),
        ],
    )(xw, degp)


def _tc2_body(acc_ref, y1_ref, dinv_ref, w2_ref, b1_ref, g1_ref, bb1_ref,
              y2_ref):
    s = acc_ref[0] + acc_ref[1] + y1_ref[...]
    dinv = dinv_ref[...][:, 0:1]
    kbn = g1_ref[...] * np.float32(1.0 / np.sqrt(1.0 + EPS))
    h = (s * dinv + b1_ref[...]) * kbn + bb1_ref[...]
    h = jnp.maximum(h, 0.0)
    y2_ref[...] = jnp.dot(h, w2_ref[...],
                          preferred_element_type=jnp.float32) * dinv


def _tc2_call(acc, y1, dinv, w2, b1, g1, bb1):
    n, dh = y1.shape
    grid = n // _BLK
    vec = lambda a: a.reshape(1, -1)
    return pl.pallas_call(
        _tc2_body,
        grid=(grid,),
        in_specs=[
            pl.BlockSpec((NC, _BLK, dh), lambda i: (0, i, 0)),
            pl.BlockSpec((_BLK, dh), lambda i: (i, 0)),
            pl.BlockSpec((_BLK, L), lambda i: (i, 0)),
            pl.BlockSpec((dh, dh), lambda i: (0, 0)),
            pl.BlockSpec((1, dh), lambda i: (0, 0)),
            pl.BlockSpec((1, dh), lambda i: (0, 0)),
            pl.BlockSpec((1, dh), lambda i: (0, 0)),
        ],
        out_specs=pl.BlockSpec((_BLK, dh), lambda i: (i, 0)),
        out_shape=jax.ShapeDtypeStruct((n, dh), jnp.float32),
    )(acc, y1, dinv, w2, vec(b1), vec(g1), vec(bb1))


def _tc3_body(n_graphs, acc_ref, y2_ref, dinv_ref, batch_ref, b2_ref, g2_ref,
              bb2_ref, wout_ref, bout_ref, out_ref, pooled_acc, cnt_acc):
    i = pl.program_id(0)
    s = acc_ref[0] + acc_ref[1] + y2_ref[...]
    dinv = dinv_ref[...][:, 0:1]
    kbn = g2_ref[...] * np.float32(1.0 / np.sqrt(1.0 + EPS))
    h = (s * dinv + b2_ref[...]) * kbn + bb2_ref[...]
    h = jnp.maximum(h, 0.0)
    gids = lax.broadcasted_iota(jnp.int32, (1, n_graphs), 1)
    onehot = (batch_ref[...] == gids).astype(jnp.float32)
    dh = h.shape[1]
    pooled_p = lax.dot_general(onehot, h, (((0,), (0,)), ((), ())),
                               preferred_element_type=jnp.float32)
    cnt_p = lax.dot_general(onehot, jnp.ones_like(h), (((0,), (0,)), ((), ())),
                            preferred_element_type=jnp.float32)

    @pl.when(i == 0)
    def _():
        pooled_acc[...] = jnp.zeros_like(pooled_acc)
        cnt_acc[...] = jnp.zeros_like(cnt_acc)

    pooled_acc[...] += pooled_p
    cnt_acc[...] += cnt_p

    @pl.when(i == pl.num_programs(0) - 1)
    def _():
        pooled = pooled_acc[...] / jnp.maximum(cnt_acc[...], 1.0)
        out_ref[...] = jnp.dot(pooled, wout_ref[...],
                               preferred_element_type=jnp.float32) + bout_ref[...]


def _tc3_call(acc, y2, dinv, batch, b2, g2, bb2, wout, bout, n_graphs):
    n, dh = y2.shape
    dout = wout.shape[1]
    grid = n // _BLK
    vec = lambda a: a.reshape(1, -1)
    body = functools.partial(_tc3_body, n_graphs)
    return pl.pallas_call(
        body,
        grid=(grid,),
        in_specs=[
            pl.BlockSpec((NC, _BLK, dh), lambda i: (0, i, 0)),
            pl.BlockSpec((_BLK, dh), lambda i: (i, 0)),
            pl.BlockSpec((_BLK, L), lambda i: (i, 0)),
            pl.BlockSpec((_BLK, 1), lambda i: (i, 0)),
            pl.BlockSpec((1, dh), lambda i: (0, 0)),
            pl.BlockSpec((1, dh), lambda i: (0, 0)),
            pl.BlockSpec((1, dh), lambda i: (0, 0)),
            pl.BlockSpec((dh, dout), lambda i: (0, 0)),
            pl.BlockSpec((1, dout), lambda i: (0, 0)),
        ],
        out_specs=pl.BlockSpec((n_graphs, dout), lambda i: (0, 0)),
        out_shape=jax.ShapeDtypeStruct((n_graphs, dout), jnp.float32),
        scratch_shapes=[
            pltpu.VMEM((n_graphs, dh), jnp.float32),
            pltpu.VMEM((n_graphs, dh), jnp.float32),
        ],
    )(acc, y2, dinv, batch.reshape(-1, 1), vec(b2), vec(g2), vec(bb2),
      wout, bout.reshape(1, -1))


def kernel(x, edge_index, edge_attr, batch, W1, b1, bn1_g, bn1_b,
           W2, b2, bn2_g, bn2_b, Wout, bout):
    n = x.shape[0]
    g = 16
    edge_index = edge_index.astype(jnp.int32)
    batch = batch.astype(jnp.int32)
    row = edge_index[0]
    col = edge_index[1]

    degp = _deg_call(col, edge_attr, n)
    xw = _tc1a_call(x, W1)
    y1, dinv = _tc1b_call(xw, degp)
    acc1 = _spmm_call(row, col, edge_attr, y1)
    y2 = _tc2_call(acc1, y1, dinv, W2, b1, bn1_g, bn1_b)
    acc2 = _spmm_call(row, col, edge_attr, y2)
    return _tc3_call(acc2, y2, dinv, batch, b2, bn2_g, bn2_b,
                     Wout, bout, g)
